# baseline (speedup 1.0000x reference)
import functools
import jax
import jax.numpy as jnp
from jax import lax
from jax.experimental import pallas as pl
from jax.experimental.pallas import tpu as pltpu
from jax.experimental.pallas import tpu_sc as plsc

NC = 2
NS = 16
NW = NC * NS
NBLK = 16


def _lane_min(v):
    lanes = lax.iota(jnp.int32, 16)
    for sh in (8, 4, 2, 1):
        idx = lanes ^ sh
        v = jnp.minimum(v, v.at[idx].get(mode="promise_in_bounds"))
    return v[0]


def _one_direction(full_v, slice_v, n_slice, m_full):
    inf = jnp.float32(3.0e38)

    def outer(ib, tot):
        n0 = ib * NBLK
        xv = slice_v[0, pl.ds(n0, NBLK)]
        yv = slice_v[1, pl.ds(n0, NBLK)]
        zv = slice_v[2, pl.ds(n0, NBLK)]
        qv = slice_v[3, pl.ds(n0, NBLK)]
        xs = [xv[j] for j in range(NBLK)]
        ys = [yv[j] for j in range(NBLK)]
        zs = [zv[j] for j in range(NBLK)]
        qs = [qv[j] for j in range(NBLK)]

        def inner(mi, accs):
            base = mi * 16
            tx = full_v[0, pl.ds(base, 16)]
            ty = full_v[1, pl.ds(base, 16)]
            tz = full_v[2, pl.ds(base, 16)]
            tq = full_v[3, pl.ds(base, 16)]
            new = []
            for j in range(NBLK):
                e = tq + xs[j] * tx + ys[j] * ty + zs[j] * tz
                new.append(jnp.minimum(accs[j], e))
            return tuple(new)

        accs = lax.fori_loop(
            0, m_full // 16, inner, tuple(jnp.full((16,), inf) for _ in range(NBLK))
        )
        for j in range(NBLK):
            tot = tot + jnp.maximum(_lane_min(accs[j]) + qs[j], 0.0)
        return tot

    return lax.fori_loop(0, n_slice // NBLK, outer, jnp.float32(0.0))


def make_sc_chamfer(B, N, M):
    n_sl = N // 4
    m_sl = M // 4
    mesh = plsc.VectorSubcoreMesh(core_axis_name="c", subcore_axis_name="s")

    @functools.partial(
        pl.kernel,
        mesh=mesh,
        out_type=jax.ShapeDtypeStruct((NW, 16), jnp.float32),
        scratch_types=[
            pltpu.VMEM((4, M), jnp.float32),
            pltpu.VMEM((4, n_sl), jnp.float32),
            pltpu.VMEM((16,), jnp.float32),
        ],
    )
    def k(srcA, tgtA, out_hbm, full_v, slice_v, out_v):
        c = lax.axis_index("c")
        s = lax.axis_index("s")
        wid = s * NC + c
        b = wid // 4
        sl = wid % 4

        pltpu.sync_copy(tgtA.at[b], full_v)
        pltpu.sync_copy(srcA.at[b, :, pl.ds(sl * n_sl, n_sl)], slice_v)
        sum_a = _one_direction(full_v, slice_v, n_sl, M)

        pltpu.sync_copy(srcA.at[b], full_v)
        pltpu.sync_copy(tgtA.at[b, :, pl.ds(sl * m_sl, m_sl)], slice_v)
        sum_b = _one_direction(full_v, slice_v, m_sl, N)

        total = (sum_a * jnp.float32(1.0 / N) + sum_b * jnp.float32(1.0 / M)) * jnp.float32(1.0 / B)
        out_v[...] = jnp.full((16,), total * jnp.float32(1.0 / 16.0))
        pltpu.sync_copy(out_v, out_hbm.at[wid])

    return k


@jax.jit
def kernel(src_points, tgt_points):
    B, N, D = src_points.shape
    M = tgt_points.shape[1]

    sq_s = jnp.sum(src_points * src_points, axis=-1)
    sq_t = jnp.sum(tgt_points * tgt_points, axis=-1)
    srcT = jnp.transpose(src_points, (0, 2, 1))
    tgtT = jnp.transpose(tgt_points, (0, 2, 1))
    srcA = jnp.concatenate([-2.0 * srcT, sq_s[:, None, :]], axis=1)
    tgtA = jnp.concatenate([tgtT, sq_t[:, None, :]], axis=1)

    out = make_sc_chamfer(B, N, M)(srcA, tgtA)
    return jnp.sum(out)


# speedup vs baseline: 11.9744x; 11.9744x over previous
import jax
import jax.numpy as jnp
from jax import lax
from jax.experimental import pallas as pl
from jax.experimental.pallas import tpu as pltpu


def _chamfer_body(aug_ref, out_ref):
    B, _, NM = aug_ref.shape
    n = NM // 2
    m = NM - n
    total = jnp.float32(0.0)
    for b in range(B):
        srcT_aug = aug_ref[b, :, :n]
        tgt_aug = aug_ref[b, :, n:]
        d2 = lax.dot_general(
            srcT_aug, tgt_aug, (((0,), (0,)), ((), ())),
            preferred_element_type=jnp.float32,
        )
        rowmin = jnp.min(d2, axis=1, keepdims=True)
        colmin = jnp.min(d2, axis=0, keepdims=True)
        total = total + (
            jnp.sum(jnp.maximum(rowmin, 0.0)) / n
            + jnp.sum(jnp.maximum(colmin, 0.0)) / m
        )
    out_ref[0, 0] = total / B


@jax.jit
def kernel(src_points, tgt_points):
    B, N, D = src_points.shape
    M = tgt_points.shape[1]

    sq_s = jnp.sum(src_points * src_points, axis=-1, keepdims=True)
    sq_t = jnp.sum(tgt_points * tgt_points, axis=-1, keepdims=True)
    ones_s = jnp.ones((B, N, 1), jnp.float32)
    ones_t = jnp.ones((B, M, 1), jnp.float32)
    src_aug = jnp.concatenate(
        [-2.0 * src_points, ones_s, sq_s, jnp.zeros((B, N, 3), jnp.float32)], axis=-1
    )
    tgt_aug = jnp.concatenate(
        [tgt_points, sq_t, ones_t, jnp.zeros((B, M, 3), jnp.float32)], axis=-1
    )
    all_aug = jnp.transpose(jnp.concatenate([src_aug, tgt_aug], axis=1), (0, 2, 1))

    out = pl.pallas_call(
        _chamfer_body,
        out_specs=pl.BlockSpec(memory_space=pltpu.SMEM),
        out_shape=jax.ShapeDtypeStruct((1, 1), jnp.float32),
    )(all_aug)
    return out[0, 0]
